# baseline (speedup 1.0000x reference)
import functools

import jax
import jax.numpy as jnp
from jax import lax
from jax.experimental import pallas as pl
from jax.experimental.pallas import tpu as pltpu
from jax.experimental.pallas import tpu_sc as plsc


def _sc_copy(nrows, d, dtype):
    info = plsc.get_sparse_core_info()
    nc, ns = info.num_cores, info.num_subcores
    nw = nc * ns
    rows_w = nrows // nw
    mesh = plsc.VectorSubcoreMesh(core_axis_name="c", subcore_axis_name="s")

    @functools.partial(
        pl.kernel,
        mesh=mesh,
        out_type=jax.ShapeDtypeStruct((nrows, d), dtype),
        scratch_types=[pltpu.SemaphoreType.DMA],
    )
    def k(x_hbm, out_hbm, sem):
        wid = lax.axis_index("s") * nc + lax.axis_index("c")
        base = wid * rows_w
        pltpu.async_copy(
            x_hbm.at[pl.ds(base, rows_w)],
            out_hbm.at[pl.ds(base, rows_w)],
            sem,
        ).wait()

    return k


def _mask_body(mask_ref):
    mask_ref[...] = jnp.ones(mask_ref.shape, dtype=jnp.bool_)


def kernel(bags):
    b, s, d = bags.shape
    flat = bags.reshape(b * s, d)
    padded = _sc_copy(b * s, d, bags.dtype)(flat).reshape(b, s, d)
    mask = pl.pallas_call(
        _mask_body,
        out_shape=jax.ShapeDtypeStruct((b, s), jnp.bool_),
    )()
    return (padded, mask)


# speedup vs baseline: 35.6245x; 35.6245x over previous
import functools

import jax
import jax.numpy as jnp
from jax import lax
from jax.experimental import pallas as pl
from jax.experimental.pallas import tpu as pltpu
from jax.experimental.pallas import tpu_sc as plsc

_CH = 64


def _sc_copy(nrows, d, dtype):
    info = plsc.get_sparse_core_info()
    nc, ns = info.num_cores, info.num_subcores
    nw = nc * ns
    rows_w = nrows // nw
    ch = min(_CH, rows_w)
    nch = rows_w // ch
    mesh = plsc.VectorSubcoreMesh(core_axis_name="c", subcore_axis_name="s")

    @functools.partial(
        pl.kernel,
        mesh=mesh,
        out_type=jax.ShapeDtypeStruct((nrows, d), dtype),
        scratch_types=[
            pltpu.VMEM((2, ch, d), dtype),
            pltpu.SemaphoreType.DMA,
            pltpu.SemaphoreType.DMA,
            pltpu.SemaphoreType.DMA,
            pltpu.SemaphoreType.DMA,
        ],
    )
    def k(x_hbm, out_hbm, buf, in0, in1, out0, out1):
        wid = lax.axis_index("s") * nc + lax.axis_index("c")
        base = wid * rows_w
        insems = (in0, in1)
        outsems = (out0, out1)

        def in_copy(j):
            b = j % 2
            return pltpu.make_async_copy(
                x_hbm.at[pl.ds(base + j * ch, ch)], buf.at[b], insems[b])

        def out_copy(j):
            b = j % 2
            return pltpu.make_async_copy(
                buf.at[b], out_hbm.at[pl.ds(base + j * ch, ch)], outsems[b])

        in_copy(0).start()
        for j in range(nch):
            if j + 1 < nch:
                if j >= 1:
                    out_copy(j - 1).wait()
                in_copy(j + 1).start()
            in_copy(j).wait()
            out_copy(j).start()
        for j in range(max(0, nch - 2), nch):
            out_copy(j).wait()

    return k


def _mask_body(mask_ref):
    mask_ref[...] = jnp.ones(mask_ref.shape, dtype=jnp.bool_)


def kernel(bags):
    b, s, d = bags.shape
    flat = bags.reshape(b * s, d)
    padded = _sc_copy(b * s, d, bags.dtype)(flat).reshape(b, s, d)
    mask = pl.pallas_call(
        _mask_body,
        out_shape=jax.ShapeDtypeStruct((b, s), jnp.bool_),
    )()
    return (padded, mask)


# speedup vs baseline: 47.6565x; 1.3377x over previous
import jax
import jax.numpy as jnp
from jax.experimental import pallas as pl
from jax.experimental.pallas import tpu as pltpu

_CHUNK_ROWS = 2048
_NBUF = 10
_WSLACK = 5


def _ring_body(x_ref, out_ref, mask_ref, buf, insem, outsem):
    n = x_ref.shape[0]
    c = min(_CHUNK_ROWS, n)
    nch = n // c

    def in_copy(j):
        b = j % _NBUF
        return pltpu.make_async_copy(
            x_ref.at[pl.ds(j * c, c)], buf.at[b], insem.at[b])

    def out_copy(j):
        b = j % _NBUF
        return pltpu.make_async_copy(
            buf.at[b], out_ref.at[pl.ds(j * c, c)], outsem.at[b])

    for j in range(min(_NBUF, nch)):
        in_copy(j).start()
    mask_ref[...] = jnp.ones(mask_ref.shape, dtype=jnp.bool_)
    for i in range(nch):
        in_copy(i).wait()
        out_copy(i).start()
        if i >= _WSLACK and (i - _WSLACK) + _NBUF < nch:
            out_copy(i - _WSLACK).wait()
            in_copy((i - _WSLACK) + _NBUF).start()
    for i in range(max(0, nch - _NBUF), nch):
        out_copy(i).wait()


def kernel(bags):
    b, s, d = bags.shape
    n = b * s
    flat = bags.reshape(n, d)
    c = min(_CHUNK_ROWS, n)
    padded, mask = pl.pallas_call(
        _ring_body,
        in_specs=[pl.BlockSpec(memory_space=pl.ANY)],
        out_specs=(
            pl.BlockSpec(memory_space=pl.ANY),
            pl.BlockSpec(memory_space=pltpu.MemorySpace.VMEM),
        ),
        out_shape=(
            jax.ShapeDtypeStruct((n, d), bags.dtype),
            jax.ShapeDtypeStruct((b, s), jnp.bool_),
        ),
        scratch_shapes=[
            pltpu.VMEM((_NBUF, c, d), bags.dtype),
            pltpu.SemaphoreType.DMA((_NBUF,)),
            pltpu.SemaphoreType.DMA((_NBUF,)),
        ],
    )(flat)
    return (padded.reshape(b, s, d), mask)


# speedup vs baseline: 48.0731x; 1.0087x over previous
import jax
import jax.numpy as jnp
from jax.experimental import pallas as pl
from jax.experimental.pallas import tpu as pltpu

_SLOT_ROWS = 4096
_NBUF = 5
_WSLACK = 2


def _chunk_schedule(n):
    if n < 2 * _SLOT_ROWS:
        return [n]
    ramp = [512, 512, 1024, 2048]
    head = [r for r in ramp]
    tail = [r for r in reversed(ramp)]
    mid_total = n - sum(head) - sum(tail)
    if mid_total < 0 or mid_total % _SLOT_ROWS != 0:
        return [_SLOT_ROWS] * (n // _SLOT_ROWS) + (
            [n % _SLOT_ROWS] if n % _SLOT_ROWS else [])
    return head + [_SLOT_ROWS] * (mid_total // _SLOT_ROWS) + tail


def _ring_body(x_ref, out_ref, mask_ref, buf, insem, outsem):
    n = x_ref.shape[0]
    sizes = _chunk_schedule(n)
    starts = []
    acc = 0
    for sz in sizes:
        starts.append(acc)
        acc += sz
    nch = len(sizes)

    def in_copy(j):
        b = j % _NBUF
        return pltpu.make_async_copy(
            x_ref.at[pl.ds(starts[j], sizes[j])],
            buf.at[b, pl.ds(0, sizes[j])],
            insem.at[b],
        )

    def out_copy(j):
        b = j % _NBUF
        return pltpu.make_async_copy(
            buf.at[b, pl.ds(0, sizes[j])],
            out_ref.at[pl.ds(starts[j], sizes[j])],
            outsem.at[b],
        )

    for j in range(min(_NBUF, nch)):
        in_copy(j).start()
    mask_ref[...] = jnp.ones(mask_ref.shape, dtype=jnp.bool_)
    for i in range(nch):
        in_copy(i).wait()
        out_copy(i).start()
        if i >= _WSLACK and (i - _WSLACK) + _NBUF < nch:
            out_copy(i - _WSLACK).wait()
            in_copy((i - _WSLACK) + _NBUF).start()
    for i in range(max(0, nch - _NBUF), nch):
        out_copy(i).wait()


def kernel(bags):
    b, s, d = bags.shape
    n = b * s
    flat = bags.reshape(n, d)
    slot = min(_SLOT_ROWS, n)
    padded, mask = pl.pallas_call(
        _ring_body,
        in_specs=[pl.BlockSpec(memory_space=pl.ANY)],
        out_specs=(
            pl.BlockSpec(memory_space=pl.ANY),
            pl.BlockSpec(memory_space=pltpu.MemorySpace.VMEM),
        ),
        out_shape=(
            jax.ShapeDtypeStruct((n, d), bags.dtype),
            jax.ShapeDtypeStruct((b, s), jnp.bool_),
        ),
        scratch_shapes=[
            pltpu.VMEM((_NBUF, slot, d), bags.dtype),
            pltpu.SemaphoreType.DMA((_NBUF,)),
            pltpu.SemaphoreType.DMA((_NBUF,)),
        ],
    )(flat)
    return (padded.reshape(b, s, d), mask)
